# baseline (speedup 1.0000x reference)
import functools

import jax
import jax.numpy as jnp
from jax import lax
from jax.experimental import pallas as pl
from jax.experimental.pallas import tpu as pltpu
from jax.experimental.pallas import tpu_sc as plsc

_EMBED = 64
_BATCH = 16384
_NC = 2
_NS = 16
_NW = _NC * _NS
_BPW = _BATCH // _NW
_CH = 128
_NCH = _BPW // _CH

_BLK = 512


def _repack_params(n_rows):
    g = -(-n_rows // (2 * _BLK)) + 1
    w2_rows = g * _BLK
    right_off = g - 1
    split = right_off * _BLK
    assert split <= w2_rows and split + w2_rows >= n_rows
    return g, w2_rows, right_off, split


def _make_producer(n_rows):
    g, w2_rows, right_off, _ = _repack_params(n_rows)
    last_in_blk = -(-n_rows // _BLK) - 1

    def body(xl_ref, xr_ref, out_ref):
        eye = jnp.eye(_EMBED, dtype=jnp.float32)
        dn = (((0,), (0,)), ((), ()))
        out_ref[:, 0:_EMBED] = lax.dot_general(
            xl_ref[...], eye, dn, preferred_element_type=jnp.float32)
        out_ref[:, _EMBED:128] = lax.dot_general(
            xr_ref[...], eye, dn, preferred_element_type=jnp.float32)

    return pl.pallas_call(
        body,
        grid=(g,),
        in_specs=[
            pl.BlockSpec((_EMBED, _BLK), lambda i: (0, i)),
            pl.BlockSpec(
                (_EMBED, _BLK),
                lambda i, _o=right_off, _m=last_in_blk:
                    (0, jnp.minimum(i + _o, _m)),
            ),
        ],
        out_specs=pl.BlockSpec((_BLK, 128), lambda i: (i, 0)),
        out_shape=jax.ShapeDtypeStruct((w2_rows, 128), jnp.float32),
    )


def _make_gather(n_rows):
    _, w2_rows, _, split = _repack_params(n_rows)

    mesh = plsc.VectorSubcoreMesh(core_axis_name="c", subcore_axis_name="s")

    @functools.partial(
        pl.kernel,
        out_type=jax.ShapeDtypeStruct((_BATCH, 128), jnp.float32),
        mesh=mesh,
        compiler_params=pltpu.CompilerParams(needs_layout_passes=False),
        scratch_types=[
            pltpu.VMEM((_BPW,), jnp.int32),
            pltpu.VMEM((_BPW,), jnp.int32),
            pltpu.VMEM((_CH, 128), jnp.float32),
            pltpu.VMEM((_CH, 128), jnp.float32),
            pltpu.VMEM((_CH, 128), jnp.float32),
            pltpu.VMEM((_CH, 128), jnp.float32),
            pltpu.SemaphoreType.DMA,
            pltpu.SemaphoreType.DMA,
        ],
    )
    def gather(nids, w2, out, idxbuf, idx2, pb0, pb1, rb0, rb1, semg, semo):
        wid = lax.axis_index("s") * _NC + lax.axis_index("c")
        base = wid * _BPW
        lanes = lax.iota(jnp.int32, 16)

        pltpu.sync_copy(nids.at[pl.ds(base, _BPW)], idxbuf)

        def vmap_body(i, _):
            v = idxbuf[pl.ds(i * 16, 16)]
            idx2[pl.ds(i * 16, 16)] = jnp.where(v < split, v, v - split)
            return 0
        lax.fori_loop(0, _BPW // 16, vmap_body, 0, unroll=False)

        pairbufs = (pb0, pb1)
        rowbufs = (rb0, rb1)

        def fire(ck):
            return pltpu.async_copy(
                w2.at[idx2.at[pl.ds(ck * _CH, _CH)]], pairbufs[ck % 2], semg)

        gathers = [fire(0), fire(1)]
        out_waits = []
        for ck in range(_NCH):
            gathers[ck].wait()
            pb = pairbufs[ck % 2]
            rb = rowbufs[ck % 2]
            if ck >= 2:
                out_waits[ck - 2].wait()
            for kk in range(_CH // 16):
                kv = kk * 16 + lanes
                v16 = idxbuf[pl.ds(ck * _CH + kk * 16, 16)]
                off16 = jnp.where(v16 < split, 0, _EMBED).astype(jnp.int32)
                zeros = jnp.zeros((16,), jnp.int32)

                def w_body(i, _, kv=kv, off16=off16, zeros=zeros,
                           pb=pb, rb=rb):
                    for dw in range(8):
                        w = i * 8 + dw
                        vals = plsc.load_gather(pb, [kv, off16 + w])
                        plsc.store_scatter(rb, [kv, zeros + w], vals)
                    return 0
                lax.fori_loop(0, _EMBED // 8, w_body, 0, unroll=False)
            if ck + 2 < _NCH:
                gathers.append(fire(ck + 2))
            out_waits.append(pltpu.async_copy(
                rb, out.at[pl.ds(base + ck * _CH, _CH), :], semo))
        for ww in out_waits[-2:]:
            ww.wait()

    return gather


_N_USER = 1000000
_N_ITEM = 100000
_N_CAT = 1000

_prod_user = _make_producer(_N_USER)
_prod_item = _make_producer(_N_ITEM)
_prod_cat = _make_producer(_N_CAT)
_gath_user = _make_gather(_N_USER)
_gath_item = _make_gather(_N_ITEM)
_gath_cat = _make_gather(_N_CAT)


def kernel(nids_user, nids_item, nids_cat, W_user, W_item, W_cat):
    w2u = _prod_user(W_user.T, W_user.T)
    w2i = _prod_item(W_item.T, W_item.T)
    w2c = _prod_cat(W_cat.T, W_cat.T)
    ou = _gath_user(nids_user.astype(jnp.int32), w2u)
    oi = _gath_item(nids_item.astype(jnp.int32), w2i)
    oc = _gath_cat(nids_cat.astype(jnp.int32), w2c)
    return (ou[:, :_EMBED], oi[:, :_EMBED], oc[:, :_EMBED])


# speedup vs baseline: 1.4856x; 1.4856x over previous
import functools

import jax
import jax.numpy as jnp
from jax import lax
from jax.experimental import pallas as pl
from jax.experimental.pallas import tpu as pltpu
from jax.experimental.pallas import tpu_sc as plsc

_EMBED = 64
_BATCH = 16384
_NC = 2
_NS = 16
_NW = _NC * _NS
_BPW = _BATCH // _NW
_CH = 128
_NCH = _BPW // _CH

_BLK = 1024


def _repack_params(n_rows):
    g = -(-n_rows // (2 * _BLK)) + 1
    w2_rows = g * _BLK
    right_off = g - 1
    split = right_off * _BLK
    assert split <= w2_rows and split + w2_rows >= n_rows
    return g, w2_rows, right_off, split


def _make_producer(n_rows):
    g, w2_rows, right_off, _ = _repack_params(n_rows)
    last_in_blk = -(-n_rows // _BLK) - 1

    def body(xl_ref, xr_ref, out_ref):
        out_ref[:, 0:_EMBED] = jnp.transpose(xl_ref[...])
        out_ref[:, _EMBED:128] = jnp.transpose(xr_ref[...])

    return pl.pallas_call(
        body,
        grid=(g,),
        in_specs=[
            pl.BlockSpec((_EMBED, _BLK), lambda i: (0, i)),
            pl.BlockSpec(
                (_EMBED, _BLK),
                lambda i, _o=right_off, _m=last_in_blk:
                    (0, jnp.minimum(i + _o, _m)),
            ),
        ],
        out_specs=pl.BlockSpec((_BLK, 128), lambda i: (i, 0)),
        out_shape=jax.ShapeDtypeStruct((w2_rows, 128), jnp.float32),
    )


def _make_gather(n_rows):
    _, w2_rows, _, split = _repack_params(n_rows)

    mesh = plsc.VectorSubcoreMesh(core_axis_name="c", subcore_axis_name="s")

    @functools.partial(
        pl.kernel,
        out_type=jax.ShapeDtypeStruct((_BATCH, 128), jnp.float32),
        mesh=mesh,
        compiler_params=pltpu.CompilerParams(needs_layout_passes=False),
        scratch_types=[
            pltpu.VMEM((_BPW,), jnp.int32),
            pltpu.VMEM((_BPW,), jnp.int32),
            pltpu.VMEM((_CH, 128), jnp.float32),
            pltpu.VMEM((_CH, 128), jnp.float32),
            pltpu.VMEM((_CH, 128), jnp.float32),
            pltpu.VMEM((_CH, 128), jnp.float32),
            pltpu.SemaphoreType.DMA,
            pltpu.SemaphoreType.DMA,
        ],
    )
    def gather(nids, w2, out, idxbuf, idx2, pb0, pb1, rb0, rb1, semg, semo):
        wid = lax.axis_index("s") * _NC + lax.axis_index("c")
        base = wid * _BPW
        lanes = lax.iota(jnp.int32, 16)

        pltpu.sync_copy(nids.at[pl.ds(base, _BPW)], idxbuf)

        def vmap_body(i, _):
            v = idxbuf[pl.ds(i * 16, 16)]
            idx2[pl.ds(i * 16, 16)] = jnp.where(v < split, v, v - split)
            return 0
        lax.fori_loop(0, _BPW // 16, vmap_body, 0, unroll=False)

        pairbufs = (pb0, pb1)
        rowbufs = (rb0, rb1)

        def fire(ck):
            return pltpu.async_copy(
                w2.at[idx2.at[pl.ds(ck * _CH, _CH)]], pairbufs[ck % 2], semg)

        gathers = [fire(0), fire(1)]
        out_waits = []
        for ck in range(_NCH):
            gathers[ck].wait()
            pb = pairbufs[ck % 2]
            rb = rowbufs[ck % 2]
            if ck >= 2:
                out_waits[ck - 2].wait()
            for kk in range(_CH // 16):
                kv = kk * 16 + lanes
                v16 = idxbuf[pl.ds(ck * _CH + kk * 16, 16)]
                off16 = jnp.where(v16 < split, 0, _EMBED).astype(jnp.int32)
                zeros = jnp.zeros((16,), jnp.int32)

                def w_body(i, _, kv=kv, off16=off16, zeros=zeros,
                           pb=pb, rb=rb):
                    for dw in range(8):
                        w = i * 8 + dw
                        vals = plsc.load_gather(pb, [kv, off16 + w])
                        plsc.store_scatter(rb, [kv, zeros + w], vals)
                    return 0
                lax.fori_loop(0, _EMBED // 8, w_body, 0, unroll=False)
            if ck + 2 < _NCH:
                gathers.append(fire(ck + 2))
            out_waits.append(pltpu.async_copy(
                rb, out.at[pl.ds(base + ck * _CH, _CH), :], semo))
        for ww in out_waits[-2:]:
            ww.wait()

    return gather


_N_USER = 1000000
_N_ITEM = 100000
_N_CAT = 1000

_prod_user = _make_producer(_N_USER)
_prod_item = _make_producer(_N_ITEM)
_prod_cat = _make_producer(_N_CAT)
_gath_user = _make_gather(_N_USER)
_gath_item = _make_gather(_N_ITEM)
_gath_cat = _make_gather(_N_CAT)


def kernel(nids_user, nids_item, nids_cat, W_user, W_item, W_cat):
    w2u = _prod_user(W_user.T, W_user.T)
    w2i = _prod_item(W_item.T, W_item.T)
    w2c = _prod_cat(W_cat.T, W_cat.T)
    ou = _gath_user(nids_user.astype(jnp.int32), w2u)
    oi = _gath_item(nids_item.astype(jnp.int32), w2i)
    oc = _gath_cat(nids_cat.astype(jnp.int32), w2c)
    return (ou[:, :_EMBED], oi[:, :_EMBED], oc[:, :_EMBED])
